# Initial kernel scaffold; baseline (speedup 1.0000x reference)
import jax
import jax.numpy as jnp
from jax.experimental import pallas as pl


def kernel(tokens, W1, b1, Wc, bc, W2, b2):
    raise NotImplementedError("write your pallas kernel here")


# speedup vs baseline: 27.5867x; 27.5867x over previous
import functools

import jax
import jax.numpy as jnp
from jax import lax
from jax.experimental import pallas as pl
from jax.experimental.pallas import tpu as pltpu
from jax.experimental.pallas import tpu_sc as plsc

B = 128
N = 32768
SKIP = 1024
M = N - SKIP
OUT_D = 1024
L = 16
NC = 2
NS = 16
NW = NC * NS
ROWS_PER_W = B // NW
NCHUNK = 4
CHUNK = M // NCHUNK


def _sc_reduce_mlp(tokens_flat, p, w2, b2):
    mesh = plsc.VectorSubcoreMesh(
        core_axis_name="c", subcore_axis_name="s",
        num_cores=NC, num_subcores=NS)

    @functools.partial(
        pl.kernel,
        out_type=jax.ShapeDtypeStruct((B * OUT_D,), jnp.float32),
        mesh=mesh,
        scratch_types=[
            pltpu.VMEM((2, CHUNK), jnp.float32),
            pltpu.VMEM((16,), jnp.float32),
            pltpu.VMEM((OUT_D,), jnp.float32),
            pltpu.VMEM((OUT_D,), jnp.float32),
            pltpu.VMEM((OUT_D,), jnp.float32),
            pltpu.SemaphoreType.DMA,
            pltpu.SemaphoreType.DMA,
        ],
    )
    def body(tok_hbm, p_hbm, w2_hbm, b2_hbm, out_hbm,
             buf, p_v, w2_v, b2_v, orow, sem0, sem1):
        sems = (sem0, sem1)
        wid = lax.axis_index("s") * NC + lax.axis_index("c")
        base_row = wid * ROWS_PER_W

        pltpu.sync_copy(p_hbm, p_v)
        pltpu.sync_copy(w2_hbm, w2_v)
        pltpu.sync_copy(b2_hbm, b2_v)
        p_vec = p_v[pl.ds(0, L)]
        p0 = p_vec[0]
        p1 = p_vec[1]

        def chunk_start(row, k):
            off = pl.multiple_of(row * N + SKIP + k * CHUNK, 8)
            return pltpu.async_copy(
                tok_hbm.at[pl.ds(off, CHUNK)], buf.at[k % 2], sems[k % 2])

        def chunk_sum(k, acc):
            slot = k % 2

            def it(i, a):
                return a + buf[slot, pl.ds(i * L, L)]

            return lax.fori_loop(0, CHUNK // L, it, acc, unroll=8)

        for r in range(ROWS_PER_W):
            row = base_row + r
            acc = jnp.zeros((L,), jnp.float32)
            cp = chunk_start(row, 0)
            for k in range(NCHUNK):
                nxt = chunk_start(row, k + 1) if k + 1 < NCHUNK else None
                cp.wait()
                acc = chunk_sum(k, acc)
                cp = nxt
            total = acc[0]
            for lane in range(1, L):
                total = total + acc[lane]
            c = p0 + p1 * total

            def out_it(j, _):
                sl = pl.ds(j * L, L)
                orow[sl] = jnp.maximum(c * w2_v[sl] + b2_v[sl], 0.0)
                return 0

            lax.fori_loop(0, OUT_D // L, out_it, 0, unroll=8)
            pltpu.sync_copy(
                orow, out_hbm.at[pl.ds(pl.multiple_of(row * OUT_D, 8), OUT_D)])

    return body(tokens_flat, p, w2, b2)


def kernel(tokens, W1, b1, Wc, bc, W2, b2):
    p = jnp.zeros((16,), jnp.float32)
    p = p.at[0].set(Wc[0, 0] + bc[0]).at[1].set(Wc[0, 1] * (1.0 / M))
    out_flat = _sc_reduce_mlp(tokens.reshape(B * N), p, W2[:, 0], b2)
    return out_flat.reshape(B, OUT_D)


# speedup vs baseline: 32.6370x; 1.1831x over previous
import functools

import jax
import jax.numpy as jnp
from jax import lax
from jax.experimental import pallas as pl
from jax.experimental.pallas import tpu as pltpu
from jax.experimental.pallas import tpu_sc as plsc

B = 128
N = 32768
SKIP = 1024
M = N - SKIP
OUT_D = 1024
L = 16
NC = 2
NS = 16
NW = NC * NS
ROWS_PER_W = B // NW
NACC = 8
STRIDE = L * NACC
NIT = M // STRIDE


def _sc_reduce_mlp(tokens_flat, p, w2, b2):
    mesh = plsc.VectorSubcoreMesh(
        core_axis_name="c", subcore_axis_name="s",
        num_cores=NC, num_subcores=NS)

    @functools.partial(
        pl.kernel,
        out_type=jax.ShapeDtypeStruct((B * OUT_D,), jnp.float32),
        mesh=mesh,
        scratch_types=[
            pltpu.VMEM((2, M), jnp.float32),
            pltpu.VMEM((16,), jnp.float32),
            pltpu.VMEM((OUT_D,), jnp.float32),
            pltpu.VMEM((OUT_D,), jnp.float32),
            pltpu.VMEM((ROWS_PER_W, OUT_D), jnp.float32),
            pltpu.SemaphoreType.DMA,
            pltpu.SemaphoreType.DMA,
            pltpu.SemaphoreType.DMA,
        ],
    )
    def body(tok_hbm, p_hbm, w2_hbm, b2_hbm, out_hbm,
             buf, p_v, w2_v, b2_v, orows, sem0, sem1, osem):
        sems = (sem0, sem1)
        wid = lax.axis_index("s") * NC + lax.axis_index("c")
        base_row = wid * ROWS_PER_W

        pltpu.sync_copy(p_hbm, p_v)
        pltpu.sync_copy(w2_hbm, w2_v)
        pltpu.sync_copy(b2_hbm, b2_v)
        p_vec = p_v[pl.ds(0, L)]
        p0 = p_vec[0]
        p1 = p_vec[1]

        def row_start(r):
            off = pl.multiple_of((base_row + r) * N + SKIP, 8)
            return pltpu.async_copy(
                tok_hbm.at[pl.ds(off, M)], buf.at[r % 2], sems[r % 2])

        def row_sum(r):
            slot = r % 2
            zero = jnp.zeros((L,), jnp.float32)

            def it(i, accs):
                base = i * STRIDE
                return tuple(
                    a + buf[slot, pl.ds(base + j * L, L)]
                    for j, a in enumerate(accs))

            accs = plsc.parallel_loop(
                0, NIT, carry=(zero,) * NACC, unroll=2)(it)
            acc = accs[0]
            for a in accs[1:]:
                acc = acc + a
            total = acc[0]
            for lane in range(1, L):
                total = total + acc[lane]
            return total

        out_copies = []
        cp = row_start(0)
        for r in range(ROWS_PER_W):
            nxt = row_start(r + 1) if r + 1 < ROWS_PER_W else None
            cp.wait()
            c = p0 + p1 * row_sum(r)

            def out_it(j):
                sl = pl.ds(j * L, L)
                orows[r, sl] = jnp.maximum(c * w2_v[sl] + b2_v[sl], 0.0)

            plsc.parallel_loop(0, OUT_D // L, unroll=4)(out_it)
            off = pl.multiple_of((base_row + r) * OUT_D, 8)
            out_copies.append(pltpu.async_copy(
                orows.at[r], out_hbm.at[pl.ds(off, OUT_D)], osem))
            cp = nxt
        for oc in out_copies:
            oc.wait()

    return body(tokens_flat, p, w2, b2)


def kernel(tokens, W1, b1, Wc, bc, W2, b2):
    p = jnp.zeros((16,), jnp.float32)
    p = p.at[0].set(Wc[0, 0] + bc[0]).at[1].set(Wc[0, 1] * (1.0 / M))
    out_flat = _sc_reduce_mlp(tokens.reshape(B * N), p, W2[:, 0], b2)
    return out_flat.reshape(B, OUT_D)
